# baseline (speedup 1.0000x reference)
import functools
import jax
import jax.numpy as jnp
from jax import lax
from jax.experimental import pallas as pl
from jax.experimental.pallas import tpu as pltpu, tpu_sc as plsc

B = 2048
C = 128
NC = 2
NS = 16
NW = NC * NS
ROWS_PER = B // NW
RB = 4
NBLK = C // 16


def _sc_body(x_hbm, keep_hbm, mt_hbm, out_hbm, x_v, mt_v, keep_v, out_v):
    cid = lax.axis_index("c")
    sid = lax.axis_index("s")
    wid = sid * NC + cid
    base = wid * ROWS_PER
    pltpu.sync_copy(x_hbm.at[pl.ds(base, ROWS_PER)], x_v)
    pltpu.sync_copy(mt_hbm, mt_v)
    pltpu.sync_copy(keep_hbm, keep_v)

    def row_block(rb, _):
        r0 = rb * RB

        def j_step(jb, acc):
            j0 = jb * 16
            xv = [x_v[r0 + r, pl.ds(j0, 16)] for r in range(RB)]
            acc = list(acc)
            for l in range(16):
                mrow = [mt_v[j0 + l, pl.ds(16 * b, 16)] for b in range(NBLK)]
                for r in range(RB):
                    xj = xv[r][l]
                    for b in range(NBLK):
                        acc[r * NBLK + b] = acc[r * NBLK + b] + xj * mrow[b]
            return tuple(acc)

        zero = jnp.zeros((16,), jnp.float32)
        acc = lax.fori_loop(0, C // 16, j_step,
                            tuple(zero for _ in range(RB * NBLK)))
        for r in range(RB):
            for b in range(NBLK):
                sl = pl.ds(16 * b, 16)
                out_v[r0 + r, sl] = x_v[r0 + r, sl] * (
                    keep_v[sl] + 0.5 * acc[r * NBLK + b])
        return 0

    lax.fori_loop(0, ROWS_PER // RB, row_block, 0)
    pltpu.sync_copy(out_v, out_hbm.at[pl.ds(base, ROWS_PER)])


def kernel(x, keep_coeff, mix_coeff):
    n, c = x.shape
    mt = mix_coeff.reshape(c, c).T
    mesh = plsc.VectorSubcoreMesh(core_axis_name="c", subcore_axis_name="s")
    k = functools.partial(
        pl.kernel,
        mesh=mesh,
        out_type=jax.ShapeDtypeStruct((n, c), jnp.float32),
        scratch_types=[
            pltpu.VMEM((ROWS_PER, C), jnp.float32),
            pltpu.VMEM((C, C), jnp.float32),
            pltpu.VMEM((C,), jnp.float32),
            pltpu.VMEM((ROWS_PER, C), jnp.float32),
        ],
    )(_sc_body)
    return k(x, keep_coeff, mt)


# speedup vs baseline: 1.7722x; 1.7722x over previous
import functools
import jax
import jax.numpy as jnp
from jax import lax
from jax.experimental import pallas as pl
from jax.experimental.pallas import tpu as pltpu, tpu_sc as plsc

B = 2048
C = 128
NC = 2
NS = 16
NW = NC * NS
SC_ROWS = 256
SC_PER = SC_ROWS // NW
RB = 4
NBLK = C // 16


def _tc_body(x_ref, keep_ref, mix_ref, o_ref):
    xb = x_ref[...]
    y = jax.lax.dot_general(
        xb, mix_ref[...], (((1,), (1,)), ((), ())),
        preferred_element_type=jnp.float32,
    )
    o_ref[...] = xb * (keep_ref[...] + 0.5 * y)


def _sc_body(x_hbm, keep_hbm, mt_hbm, out_hbm, x_v, mt_v, keep_v, out_v):
    cid = lax.axis_index("c")
    sid = lax.axis_index("s")
    wid = sid * NC + cid
    base = wid * SC_PER
    pltpu.sync_copy(x_hbm.at[pl.ds(base, SC_PER)], x_v)
    pltpu.sync_copy(mt_hbm, mt_v)
    pltpu.sync_copy(keep_hbm, keep_v)

    def row_block(rb, _):
        r0 = rb * RB

        def j_step(jb, acc):
            j0 = jb * 16
            xv = [x_v[r0 + r, pl.ds(j0, 16)] for r in range(RB)]
            acc = list(acc)
            for l in range(16):
                mrow = [mt_v[j0 + l, pl.ds(16 * b, 16)] for b in range(NBLK)]
                for r in range(RB):
                    xj = xv[r][l]
                    for b in range(NBLK):
                        acc[r * NBLK + b] = acc[r * NBLK + b] + xj * mrow[b]
            return tuple(acc)

        zero = jnp.zeros((16,), jnp.float32)
        acc = lax.fori_loop(0, C // 16, j_step,
                            tuple(zero for _ in range(RB * NBLK)))
        for r in range(RB):
            for b in range(NBLK):
                sl = pl.ds(16 * b, 16)
                out_v[r0 + r, sl] = x_v[r0 + r, sl] * (
                    keep_v[sl] + 0.5 * acc[r * NBLK + b])
        return 0

    lax.fori_loop(0, SC_PER // RB, row_block, 0)
    pltpu.sync_copy(out_v, out_hbm.at[pl.ds(base, SC_PER)])


def kernel(x, keep_coeff, mix_coeff):
    n, c = x.shape
    mix = mix_coeff.reshape(c, c)
    keep2d = keep_coeff.reshape(1, c)
    tc_rows = n - SC_ROWS

    tc_out = pl.pallas_call(
        _tc_body,
        out_shape=jax.ShapeDtypeStruct((tc_rows, c), x.dtype),
        grid=(1,),
        in_specs=[
            pl.BlockSpec((tc_rows, c), lambda i: (0, 0)),
            pl.BlockSpec((1, c), lambda i: (0, 0)),
            pl.BlockSpec((c, c), lambda i: (0, 0)),
        ],
        out_specs=pl.BlockSpec((tc_rows, c), lambda i: (0, 0)),
    )(x[:tc_rows], keep2d, mix)

    mesh = plsc.VectorSubcoreMesh(core_axis_name="c", subcore_axis_name="s")
    sc_k = functools.partial(
        pl.kernel,
        mesh=mesh,
        out_type=jax.ShapeDtypeStruct((SC_ROWS, c), jnp.float32),
        scratch_types=[
            pltpu.VMEM((SC_PER, C), jnp.float32),
            pltpu.VMEM((C, C), jnp.float32),
            pltpu.VMEM((C,), jnp.float32),
            pltpu.VMEM((SC_PER, C), jnp.float32),
        ],
    )(_sc_body)
    sc_out = sc_k(x[tc_rows:], keep_coeff, mix.T)

    return jnp.concatenate([tc_out, sc_out], axis=0)


# speedup vs baseline: 21.5653x; 12.1687x over previous
import jax
import jax.numpy as jnp
from jax.experimental import pallas as pl


def _selfmix_kernel(x_ref, keep_ref, mix_ref, o_ref):
    xb = x_ref[...]
    y = jax.lax.dot_general(
        xb, mix_ref[...], (((1,), (1,)), ((), ())),
        preferred_element_type=jnp.float32,
    )
    o_ref[...] = xb * (keep_ref[...] + 0.5 * y)


def kernel(x, keep_coeff, mix_coeff):
    n, c = x.shape
    mix = mix_coeff.reshape(c, c)
    keep = keep_coeff.reshape(1, c)
    blk = 1024
    grid = n // blk
    return pl.pallas_call(
        _selfmix_kernel,
        out_shape=jax.ShapeDtypeStruct((n, c), x.dtype),
        grid=(grid,),
        in_specs=[
            pl.BlockSpec((blk, c), lambda i: (i, 0)),
            pl.BlockSpec((1, c), lambda i: (0, 0)),
            pl.BlockSpec((c, c), lambda i: (0, 0)),
        ],
        out_specs=pl.BlockSpec((blk, c), lambda i: (i, 0)),
    )(x, keep, mix)


# speedup vs baseline: 21.6253x; 1.0028x over previous
import jax
import jax.numpy as jnp
from jax.experimental import pallas as pl
from jax.experimental.pallas import tpu as pltpu


def _selfmix_kernel(x_ref, keep_ref, mix_ref, o_ref):
    xb = x_ref[...]
    y = jax.lax.dot_general(
        xb, mix_ref[...], (((1,), (1,)), ((), ())),
        preferred_element_type=jnp.float32,
    )
    o_ref[...] = xb * (keep_ref[...] + 0.5 * y)


def kernel(x, keep_coeff, mix_coeff):
    n, c = x.shape
    mix = mix_coeff.reshape(c, c)
    keep = keep_coeff.reshape(1, c)
    blk = 1024
    grid = n // blk
    return pl.pallas_call(
        _selfmix_kernel,
        out_shape=jax.ShapeDtypeStruct((n, c), x.dtype),
        grid=(grid,),
        in_specs=[
            pl.BlockSpec((blk, c), lambda i: (i, 0)),
            pl.BlockSpec((1, c), lambda i: (0, 0)),
            pl.BlockSpec((c, c), lambda i: (0, 0)),
        ],
        out_specs=pl.BlockSpec((blk, c), lambda i: (i, 0)),
        compiler_params=pltpu.CompilerParams(
            dimension_semantics=("parallel",)),
    )(x, keep, mix)
